# Initial kernel scaffold; baseline (speedup 1.0000x reference)
import jax
import jax.numpy as jnp
from jax.experimental import pallas as pl


def kernel(x, embd):
    raise NotImplementedError("write your pallas kernel here")


# speedup vs baseline: 1.2449x; 1.2449x over previous
import functools

import jax
import jax.numpy as jnp
from jax import lax
from jax.experimental import pallas as pl
from jax.experimental.pallas import tpu as pltpu
from jax.experimental.pallas import tpu_sc as plsc

BATCH = 16384
FIELDS = 26
DIM = 32
NUM_ROWS = BATCH * FIELDS
NC = 2
NS = 16
NW = NC * NS
ROWS_PER_W = NUM_ROWS // NW
CHUNK = 128
N_CHUNKS = ROWS_PER_W // CHUNK

_mesh = plsc.VectorSubcoreMesh(core_axis_name="c", subcore_axis_name="s")


@functools.partial(
    pl.kernel,
    mesh=_mesh,
    compiler_params=pltpu.CompilerParams(use_tc_tiling_on_sc=False),
    out_type=jax.ShapeDtypeStruct((NUM_ROWS, DIM), jnp.float32),
    scratch_types=[
        pltpu.VMEM((N_CHUNKS, CHUNK), jnp.int32),
        pltpu.VMEM((CHUNK, DIM), jnp.float32),
        pltpu.SemaphoreType.DMA,
    ],
)
def _gather_kernel(idx_hbm, table_hbm, out_hbm, idx_v, rows_v, sem):
    wid = lax.axis_index("s") * NC + lax.axis_index("c")
    base = wid * ROWS_PER_W
    pltpu.sync_copy(idx_hbm.at[wid], idx_v)

    def body(j, carry):
        pltpu.async_copy(table_hbm.at[idx_v.at[j]], rows_v, sem).wait()
        pltpu.sync_copy(rows_v, out_hbm.at[pl.ds(base + j * CHUNK, CHUNK)])
        return carry

    lax.fori_loop(0, N_CHUNKS, body, 0)


def kernel(x, embd):
    idx = x.reshape(NW, N_CHUNKS, CHUNK)
    out = _gather_kernel(idx, embd)
    return out.reshape(BATCH, FIELDS, DIM)


# speedup vs baseline: 1.3413x; 1.0774x over previous
import functools

import jax
import jax.numpy as jnp
from jax import lax
from jax.experimental import pallas as pl
from jax.experimental.pallas import tpu as pltpu
from jax.experimental.pallas import tpu_sc as plsc

BATCH = 16384
FIELDS = 26
DIM = 32
NUM_ROWS = BATCH * FIELDS
NC = 2
NS = 16
NW = NC * NS
ROWS_PER_W = NUM_ROWS // NW
CHUNK = 128
N_CHUNKS = ROWS_PER_W // CHUNK
SUP = 512
K = SUP // CHUNK
NSUP = ROWS_PER_W // SUP

_mesh = plsc.VectorSubcoreMesh(core_axis_name="c", subcore_axis_name="s")


@functools.partial(
    pl.kernel,
    mesh=_mesh,
    compiler_params=pltpu.CompilerParams(use_tc_tiling_on_sc=False),
    out_type=jax.ShapeDtypeStruct((NUM_ROWS, DIM), jnp.float32),
    scratch_types=[
        pltpu.VMEM((N_CHUNKS, CHUNK), jnp.int32),
        pltpu.VMEM((SUP, DIM), jnp.float32),
        pltpu.VMEM((SUP, DIM), jnp.float32),
        pltpu.SemaphoreType.DMA,
        pltpu.SemaphoreType.DMA,
        pltpu.SemaphoreType.DMA,
        pltpu.SemaphoreType.DMA,
    ],
)
def _gather_kernel(idx_hbm, table_hbm, out_hbm, idx_v, buf_a, buf_b,
                   ga, gb, wa, wb):
    wid = lax.axis_index("s") * NC + lax.axis_index("c")
    base = wid * ROWS_PER_W
    pltpu.sync_copy(idx_hbm.at[wid], idx_v)

    def fire_gathers(s, buf, sem):
        for k in range(K):
            pltpu.async_copy(table_hbm.at[idx_v.at[s * K + k]],
                             buf.at[pl.ds(k * CHUNK, CHUNK)], sem)

    def drain_gathers(s, buf, sem):
        for k in range(K):
            pltpu.make_async_copy(table_hbm.at[idx_v.at[s * K + k]],
                                  buf.at[pl.ds(k * CHUNK, CHUNK)], sem).wait()

    def fire_write(s, buf, sem):
        pltpu.async_copy(buf, out_hbm.at[pl.ds(base + s * SUP, SUP)], sem)

    def drain_write(s, buf, sem):
        pltpu.make_async_copy(buf, out_hbm.at[pl.ds(base + s * SUP, SUP)],
                              sem).wait()

    fire_gathers(0, buf_a, ga)

    def body(i, carry):
        s0 = 2 * i
        drain_gathers(s0, buf_a, ga)

        @pl.when(i > 0)
        def _():
            drain_write(s0 - 1, buf_b, wb)

        fire_gathers(s0 + 1, buf_b, gb)
        fire_write(s0, buf_a, wa)
        drain_gathers(s0 + 1, buf_b, gb)
        drain_write(s0, buf_a, wa)

        @pl.when(i < NSUP // 2 - 1)
        def _():
            fire_gathers(s0 + 2, buf_a, ga)

        fire_write(s0 + 1, buf_b, wb)
        return carry

    lax.fori_loop(0, NSUP // 2, body, 0)
    drain_write(NSUP - 1, buf_b, wb)


def kernel(x, embd):
    idx = x.reshape(NW, N_CHUNKS, CHUNK)
    out = _gather_kernel(idx, embd)
    return out.reshape(BATCH, FIELDS, DIM)
